# baseline (speedup 1.0000x reference)
import functools

import jax
import jax.numpy as jnp
from jax import lax
from jax.experimental import pallas as pl
from jax.experimental.pallas import tpu as pltpu
from jax.experimental.pallas import tpu_sc as plsc

_B = 16384
_D = 64
_NC = 2
_NS = 16
_NW = _NC * _NS
_BPW = _B // _NW
_CHUNK = 128
_NCHUNK = _BPW // _CHUNK
_GROUPS = _BPW // 16

_mesh = plsc.VectorSubcoreMesh(core_axis_name="c", subcore_axis_name="s")


@functools.partial(
    pl.kernel,
    mesh=_mesh,
    out_type=[
        jax.ShapeDtypeStruct((_B, _D), jnp.float32),
        jax.ShapeDtypeStruct((_B,), jnp.float32),
    ],
    scratch_types=[
        pltpu.VMEM((_NCHUNK, _CHUNK), jnp.int32),
        pltpu.VMEM((_BPW, _D), jnp.float32),
        pltpu.VMEM((_BPW,), jnp.float32),
        pltpu.SemaphoreType.DMA,
        pltpu.SemaphoreType.DMA,
    ],
    compiler_params=pltpu.CompilerParams(
        needs_layout_passes=False, use_tc_tiling_on_sc=False),
)
def _sc_gather_loss(idx_hbm, table_hbm, feat_hbm, loss_hbm,
                    idx_v, rows_v, loss_v, gsem, osem):
    wid = lax.axis_index("s") * _NC + lax.axis_index("c")
    base = wid * _BPW

    pltpu.sync_copy(idx_hbm.at[wid], idx_v)

    copies = []
    for j in range(_NCHUNK):
        copies.append(pltpu.async_copy(
            table_hbm.at[idx_v.at[j]],
            rows_v.at[pl.ds(j * _CHUNK, _CHUNK)],
            gsem))
    for c in copies:
        c.wait()

    out_copy = pltpu.async_copy(rows_v, feat_hbm.at[pl.ds(base, _BPW)], osem)

    lanes = lax.iota(jnp.int32, 16)

    def group_body(g, carry):
        row_ids = g * 16 + lanes
        acc = jnp.zeros((16,), jnp.float32)
        for c in range(_D):
            col = jnp.full((16,), c, jnp.int32)
            v = plsc.load_gather(rows_v, [row_ids, col])
            acc = acc + v * v
        loss_v[pl.ds(g * 16, 16)] = acc * (0.1 / _D)
        return carry

    lax.fori_loop(0, _GROUPS, group_body, 0)

    pltpu.sync_copy(loss_v, loss_hbm.at[pl.ds(base, _BPW)])
    out_copy.wait()


def kernel(input, kernel):
    idx = jnp.asarray(input, jnp.int32).reshape(_NW, _NCHUNK, _CHUNK)
    feat, loss = _sc_gather_loss(idx, kernel)
    return feat, loss.reshape(_B, 1)


# speedup vs baseline: 1.6907x; 1.6907x over previous
import functools

import jax
import jax.numpy as jnp
from jax import lax
from jax.experimental import pallas as pl
from jax.experimental.pallas import tpu as pltpu
from jax.experimental.pallas import tpu_sc as plsc

_B = 16384
_D = 64
_NC = 2
_NS = 16
_NW = _NC * _NS
_BPW = _B // _NW
_GROUPS = _BPW // 16

_mesh = plsc.VectorSubcoreMesh(core_axis_name="c", subcore_axis_name="s")


@functools.partial(
    pl.kernel,
    mesh=_mesh,
    out_type=[
        jax.ShapeDtypeStruct((_B, _D), jnp.float32),
        jax.ShapeDtypeStruct((_B,), jnp.float32),
    ],
    scratch_types=[
        pltpu.VMEM((_BPW,), jnp.int32),
        pltpu.VMEM((_BPW, _D), jnp.float32),
        pltpu.VMEM((_BPW,), jnp.float32),
        pltpu.SemaphoreType.DMA,
        pltpu.SemaphoreType.DMA,
    ],
    compiler_params=pltpu.CompilerParams(needs_layout_passes=False),
)
def _sc_gather_loss(idx_hbm, table_hbm, feat_hbm, loss_hbm,
                    idx_v, rows_v, loss_v, gsem, osem):
    wid = lax.axis_index("s") * _NC + lax.axis_index("c")
    base = wid * _BPW

    pltpu.sync_copy(idx_hbm.at[pl.ds(base, _BPW)], idx_v)

    def issue_body(g, carry):
        v_idx = idx_v[pl.ds(g * 16, 16)]
        for j in range(16):
            row = v_idx[j]
            pltpu.async_copy(
                table_hbm.at[pl.dslice(row, 1)],
                rows_v.at[pl.dslice(g * 16 + j, 1)],
                gsem)
        return carry

    lax.fori_loop(0, _GROUPS, issue_body, 0)

    pltpu.make_async_copy(
        feat_hbm.at[pl.ds(base, _BPW)], rows_v, gsem).wait()

    out_copy = pltpu.async_copy(rows_v, feat_hbm.at[pl.ds(base, _BPW)], osem)

    lanes = lax.iota(jnp.int32, 16)

    def group_body(g, carry):
        row_ids = g * 16 + lanes
        acc = jnp.zeros((16,), jnp.float32)
        for c in range(_D):
            col = jnp.full((16,), c, jnp.int32)
            v = plsc.load_gather(rows_v, [row_ids, col])
            acc = acc + v * v
        loss_v[pl.ds(g * 16, 16)] = acc * (0.1 / _D)
        return carry

    lax.fori_loop(0, _GROUPS, group_body, 0)

    pltpu.sync_copy(loss_v, loss_hbm.at[pl.ds(base, _BPW)])
    out_copy.wait()


def kernel(input, kernel):
    idx = jnp.asarray(input, jnp.int32)
    feat, loss = _sc_gather_loss(idx, kernel)
    return feat, loss.reshape(_B, 1)


# speedup vs baseline: 1.7010x; 1.0061x over previous
import functools

import jax
import jax.numpy as jnp
from jax import lax
from jax.experimental import pallas as pl
from jax.experimental.pallas import tpu as pltpu
from jax.experimental.pallas import tpu_sc as plsc

_B = 16384
_D = 64
_NC = 2
_NS = 16
_NW = _NC * _NS
_BPW = _B // _NW
_GROUPS = _BPW // 16

_mesh = plsc.VectorSubcoreMesh(core_axis_name="c", subcore_axis_name="s")


@functools.partial(
    pl.kernel,
    mesh=_mesh,
    out_type=[
        jax.ShapeDtypeStruct((_B, _D), jnp.float32),
        jax.ShapeDtypeStruct((_B,), jnp.float32),
    ],
    scratch_types=[
        pltpu.VMEM((_BPW,), jnp.int32),
        pltpu.VMEM((_BPW, _D), jnp.float32),
        pltpu.VMEM((_BPW,), jnp.float32),
        pltpu.SemaphoreType.DMA,
        pltpu.SemaphoreType.DMA,
    ],
    compiler_params=pltpu.CompilerParams(
        needs_layout_passes=False, use_tc_tiling_on_sc=True),
)
def _sc_gather_loss(idx_hbm, table_hbm, feat_hbm, loss_hbm,
                    idx_v, rows_v, loss_v, gsem, osem):
    wid = lax.axis_index("s") * _NC + lax.axis_index("c")
    base = wid * _BPW

    pltpu.sync_copy(idx_hbm.at[pl.ds(base, _BPW)], idx_v)

    def issue_body(g, carry):
        v_idx = idx_v[pl.ds(g * 16, 16)]
        for j in range(16):
            row = v_idx[j]
            pltpu.async_copy(
                table_hbm.at[pl.dslice(row, 1)],
                rows_v.at[pl.dslice(g * 16 + j, 1)],
                gsem)
        return carry

    lax.fori_loop(0, _GROUPS, issue_body, 0)

    pltpu.make_async_copy(
        feat_hbm.at[pl.ds(base, _BPW)], rows_v, gsem).wait()

    out_copy = pltpu.async_copy(rows_v, feat_hbm.at[pl.ds(base, _BPW)], osem)

    lanes = lax.iota(jnp.int32, 16)

    def group_body(g, carry):
        row_ids = g * 16 + lanes
        acc = jnp.zeros((16,), jnp.float32)
        for c in range(_D):
            col = jnp.full((16,), c, jnp.int32)
            v = plsc.load_gather(rows_v, [row_ids, col])
            acc = acc + v * v
        loss_v[pl.ds(g * 16, 16)] = acc * (0.1 / _D)
        return carry

    lax.fori_loop(0, _GROUPS, group_body, 0)

    pltpu.sync_copy(loss_v, loss_hbm.at[pl.ds(base, _BPW)])
    out_copy.wait()


def kernel(input, kernel):
    idx = jnp.asarray(input, jnp.int32)
    feat, loss = _sc_gather_loss(idx, kernel)
    return feat, loss.reshape(_B, 1)
